# baseline (speedup 1.0000x reference)
import jax
import jax.numpy as jnp
from jax import lax
from jax._src import config as _jax_config
from jax.experimental import pallas as pl
from jax.experimental.pallas import tpu as pltpu
from jax.experimental.pallas import tpu_sc as plsc

NC = 2
NS = 16
NW = NC * NS
L = 16
CH = 48
NB = 3


def _zq_body(x_ref, w_ref, wt_ref, z_ref, q_ref):
    z = lax.dot_general(
        x_ref[...], w_ref[...], (((1,), (1,)), ((), ())),
        preferred_element_type=jnp.float32)
    z_ref[...] = z
    q_ref[...] = lax.dot_general(
        z, wt_ref[...], (((1,), (0,)), ((), ())),
        preferred_element_type=jnp.float32)


def _zq_call(x, W_fc, wt2, bn):
    n, d = x.shape
    return pl.pallas_call(
        _zq_body,
        grid=(n // bn,),
        in_specs=[
            pl.BlockSpec((bn, d), lambda i: (i, 0)),
            pl.BlockSpec((d, d), lambda i: (0, 0)),
            pl.BlockSpec((d, 1), lambda i: (0, 0)),
        ],
        out_specs=[
            pl.BlockSpec((bn, d), lambda i: (i, 0)),
            pl.BlockSpec((bn, 1), lambda i: (i, 0)),
        ],
        out_shape=[
            jax.ShapeDtypeStruct((n, d), jnp.float32),
            jax.ShapeDtypeStruct((n, 1), jnp.float32),
        ],
    )(x, W_fc, wt2)


def _combine_body(z_ref, num_ref, den_ref, o_ref):
    den = jnp.sum(den_ref[...], axis=1)
    num = num_ref[0] + num_ref[1]
    dcol = den[:, None]
    h = jnp.where(dcol > 0.0, num / jnp.where(dcol > 0.0, dcol, 1.0), 0.0)
    o_ref[...] = z_ref[...] + h


def _combine_call(z, num, den, bn):
    n, d = z.shape
    return pl.pallas_call(
        _combine_body,
        grid=(n // bn,),
        in_specs=[
            pl.BlockSpec((bn, d), lambda i: (i, 0)),
            pl.BlockSpec((NC, bn, d), lambda i: (0, i, 0)),
            pl.BlockSpec((bn, NW), lambda i: (i, 0)),
        ],
        out_specs=pl.BlockSpec((bn, d), lambda i: (i, 0)),
        out_shape=jax.ShapeDtypeStruct((n, d), jnp.float32),
    )(z, num, den)


def _make_sc_kernel(n, d, e, nch0, nch1):
    rows_per_tile = (-(-n // NS) + 7) // 8 * 8
    np_ = NS * rows_per_tile
    nfull = rows_per_tile // CH
    rem = rows_per_tile - nfull * CH
    mesh = plsc.VectorSubcoreMesh(
        core_axis_name="c", subcore_axis_name="s",
        num_cores=NC, num_subcores=NS)

    def body(t_hbm, q_hbm, edges_hbm, z_hbm,
             num_hbm, den_hbm,
             t_v, q_v, eidx_v, w_v, dsts_v, rows_v, den_v, num_s,
             sem_e0, sem_e1, sem_e2, sem_g0, sem_g1, sem_g2,
             sem_s0, sem_s1, sem_s2):
        c = lax.axis_index("c")
        s = lax.axis_index("s")
        wid = c * NS + s
        sem_e = (sem_e0, sem_e1, sem_e2)
        sem_g = (sem_g0, sem_g1, sem_g2)
        sem_s = (sem_s0, sem_s1, sem_s2)

        pltpu.sync_copy(t_hbm, t_v)
        pltpu.sync_copy(q_hbm, q_v)

        @pl.loop(0, n, step=L)
        def _(i):
            den_v[0, pl.ds(i, L)] = jnp.zeros((L,), jnp.float32)

        @pl.loop(0, CH)
        def _(r):
            for cc in range(d // L):
                rows_v[0, r, pl.ds(cc * L, L)] = jnp.zeros((L,), jnp.float32)

        base = s * rows_per_tile
        for k in range(nfull):
            pltpu.sync_copy(rows_v.at[0], num_s.at[pl.ds(base + k * CH, CH)])
        if rem:
            pltpu.sync_copy(rows_v.at[0, pl.ds(0, rem)],
                            num_s.at[pl.ds(base + nfull * CH, rem)])
        plsc.subcore_barrier()

        for b in range(NB):
            pltpu.async_copy(edges_hbm.at[wid, b], eidx_v.at[b], sem_e[b])
        pltpu.make_async_copy(edges_hbm.at[wid, 0], eidx_v.at[0], sem_e[0]).wait()
        pltpu.async_copy(z_hbm.at[eidx_v.at[0, 0]], rows_v.at[0], sem_g[0])

        def run_pipeline(nch, ebase):
            @pl.loop(0, nch // NB)
            def _(jj):
                _chunk_block(jj, nch, ebase)

        def _chunk_block(jj, nch, ebase):
            for b in range(NB):
                j = jj * NB + b
                b1 = (b + 1) % NB
                b2 = (b + 2) % NB

                pltpu.make_async_copy(
                    z_hbm.at[eidx_v.at[b, 0]], rows_v.at[b], sem_g[b]).wait()

                @pl.when(j + 1 < nch)
                def _():
                    @pl.when(j >= 2)
                    def _():
                        pltpu.make_async_copy(
                            rows_v.at[b1], num_s.at[dsts_v.at[b1]],
                            sem_s[b1]).wait()
                    pltpu.make_async_copy(
                        edges_hbm.at[wid, j + 1], eidx_v.at[b1],
                        sem_e[b1]).wait()
                    pltpu.async_copy(
                        z_hbm.at[eidx_v.at[b1, 0]], rows_v.at[b1], sem_g[b1])

                for cc in range(CH // L):
                    sl = pl.ds(cc * L, L)
                    sv = eidx_v[b, 0, sl]
                    dv = eidx_v[b, 1, sl]
                    qs = plsc.load_gather(q_v, [sv])
                    ts = plsc.load_gather(t_v, [sv])
                    td = plsc.load_gather(t_v, [dv])
                    u = jnp.exp(qs * (-jnp.abs(ts - td)) / 500.0)
                    w = jnp.exp(u)
                    gidx = ebase + j * CH + cc * L + lax.iota(jnp.int32, L)
                    w = jnp.where(gidx < e, w, 0.0)
                    w_v[sl] = w
                    dsts_v[b, sl] = dv
                    plsc.addupdate_scatter(
                        den_v, [jnp.zeros((L,), jnp.int32), dv], w)

                @pl.loop(0, CH, unroll=2)
                def _(r):
                    wsp = plsc.load_gather(w_v, [jnp.full((L,), r, jnp.int32)])
                    for cc in range(d // L):
                        sl = pl.ds(cc * L, L)
                        rows_v[b, r, sl] = rows_v[b, r, sl] * wsp

                pltpu.async_copy(rows_v.at[b], num_s.at[dsts_v.at[b]],
                                 sem_s[b], add=True)

                @pl.when((j >= 1) & (j + 2 < nch))
                def _():
                    pltpu.async_copy(edges_hbm.at[wid, j + 2], eidx_v.at[b2],
                                     sem_e[b2])

        e0w = nch0 * CH
        e1w = nch1 * CH

        @pl.when(c == 0)
        def _():
            run_pipeline(nch0, s * e0w)

        @pl.when(c == 1)
        def _():
            run_pipeline(nch1, NS * e0w + s * e1w)

        for b in range(NB):
            pltpu.make_async_copy(
                rows_v.at[b], num_s.at[dsts_v.at[b]], sem_s[b]).wait()

        plsc.subcore_barrier()

        pltpu.sync_copy(den_v, den_hbm.at[wid])
        for k in range(nfull):
            pltpu.sync_copy(num_s.at[pl.ds(base + k * CH, CH)],
                            num_hbm.at[c, pl.ds(base + k * CH, CH)])
        if rem:
            pltpu.sync_copy(num_s.at[pl.ds(base + nfull * CH, rem)],
                            num_hbm.at[c, pl.ds(base + nfull * CH, rem)])

    return pl.kernel(
        body,
        out_type=(
            jax.ShapeDtypeStruct((NC, np_, d), jnp.float32),
            jax.ShapeDtypeStruct((NW, 1, n), jnp.float32),
        ),
        mesh=mesh,
        compiler_params=pltpu.CompilerParams(needs_layout_passes=False),
        scratch_types=[
            pltpu.VMEM((n,), jnp.float32),
            pltpu.VMEM((n,), jnp.float32),
            pltpu.VMEM((NB, 2, CH), jnp.int32),
            pltpu.VMEM((CH,), jnp.float32),
            pltpu.VMEM((NB, CH), jnp.int32),
            pltpu.VMEM((NB, CH, d), jnp.float32),
            pltpu.VMEM((1, n), jnp.float32),
            pltpu.VMEM_SHARED((np_, d), jnp.float32),
        ] + [pltpu.SemaphoreType.DMA] * (3 * NB),
    )


def kernel(x, t, W_fc, w_t, edge_index):
    with _jax_config.enable_x64(False):
        n, d = x.shape
        e = edge_index.shape[1]

        f0 = 0.36
        nch0 = max(NB, int(round(e * f0 / (NS * CH * NB))) * NB)
        rem_e = max(NS * CH * NB, e - NS * nch0 * CH)
        nch1 = -(-rem_e // (NS * CH))
        nch1 = -(-nch1 // NB) * NB
        nchm = max(nch0, nch1)
        e0 = NS * nch0 * CH
        ep = e0 + NS * nch1 * CH

        ei = edge_index.astype(jnp.int32)
        sflat = jnp.pad(ei[0], (0, ep - e))
        dflat = jnp.pad(ei[1], (0, ep - e))

        def _shape_core(a):
            a0 = jnp.pad(a[:e0].reshape(NS, nch0, CH),
                         ((0, 0), (0, nchm - nch0), (0, 0)))
            a1 = jnp.pad(a[e0:].reshape(NS, nch1, CH),
                         ((0, 0), (0, nchm - nch1), (0, 0)))
            return jnp.concatenate([a0, a1], axis=0)

        edges = jnp.stack([_shape_core(sflat), _shape_core(dflat)],
                          axis=2)

        bn = 2000 if n % 2000 == 0 else n
        z, q2 = _zq_call(x, W_fc, w_t.reshape(d, 1), bn)
        q = q2.reshape(n)

        num, den = _make_sc_kernel(n, d, e, nch0, nch1)(t, q, edges, z)
        return _combine_call(z, num[:, :n, :], den.reshape(NW, n).T, bn)


# speedup vs baseline: 1.2054x; 1.2054x over previous
import jax
import jax.numpy as jnp
from jax import lax
from jax._src import config as _jax_config
from jax.experimental import pallas as pl
from jax.experimental.pallas import tpu as pltpu
from jax.experimental.pallas import tpu_sc as plsc

NC = 2
NS = 16
NW = NC * NS
L = 16
CH = 48
NB = 3


def _zq_body(x_ref, w_ref, wt_ref, z_ref, q_ref):
    z = lax.dot_general(
        x_ref[...], w_ref[...], (((1,), (1,)), ((), ())),
        preferred_element_type=jnp.float32)
    z_ref[...] = z
    q_ref[...] = lax.dot_general(
        z, wt_ref[...], (((1,), (0,)), ((), ())),
        preferred_element_type=jnp.float32)


def _zq_call(x, W_fc, wt2, bn):
    n, d = x.shape
    return pl.pallas_call(
        _zq_body,
        grid=(n // bn,),
        in_specs=[
            pl.BlockSpec((bn, d), lambda i: (i, 0)),
            pl.BlockSpec((d, d), lambda i: (0, 0)),
            pl.BlockSpec((d, 1), lambda i: (0, 0)),
        ],
        out_specs=[
            pl.BlockSpec((bn, d), lambda i: (i, 0)),
            pl.BlockSpec((bn, 1), lambda i: (i, 0)),
        ],
        out_shape=[
            jax.ShapeDtypeStruct((n, d), jnp.float32),
            jax.ShapeDtypeStruct((n, 1), jnp.float32),
        ],
    )(x, W_fc, wt2)


def _combine_body(z_ref, num_ref, den_ref, o_ref):
    den = jnp.sum(den_ref[...], axis=1)
    num = num_ref[0] + num_ref[1]
    dcol = den[:, None]
    h = jnp.where(dcol > 0.0, num / jnp.where(dcol > 0.0, dcol, 1.0), 0.0)
    o_ref[...] = z_ref[...] + h


def _combine_call(z, num, den, bn):
    n, d = z.shape
    return pl.pallas_call(
        _combine_body,
        grid=(n // bn,),
        in_specs=[
            pl.BlockSpec((bn, d), lambda i: (i, 0)),
            pl.BlockSpec((NC, bn, d), lambda i: (0, i, 0)),
            pl.BlockSpec((bn, NW), lambda i: (i, 0)),
        ],
        out_specs=pl.BlockSpec((bn, d), lambda i: (i, 0)),
        out_shape=jax.ShapeDtypeStruct((n, d), jnp.float32),
    )(z, num, den)


def _make_sc_kernel(n, d, e, nch0, nch1):
    rows_per_tile = (-(-n // NS) + 7) // 8 * 8
    np_ = NS * rows_per_tile
    nfull = rows_per_tile // CH
    rem = rows_per_tile - nfull * CH
    mesh = plsc.VectorSubcoreMesh(
        core_axis_name="c", subcore_axis_name="s",
        num_cores=NC, num_subcores=NS)

    def body(t_hbm, q_hbm, edges_hbm, z_hbm,
             num_hbm, den_hbm,
             t_v, q_v, eidx_v, w_v, dsts_v, rows_v, den_v, num_s,
             sem_e0, sem_e1, sem_e2, sem_g0, sem_g1, sem_g2,
             sem_s0, sem_s1, sem_s2):
        c = lax.axis_index("c")
        s = lax.axis_index("s")
        wid = c * NS + s
        sem_e = (sem_e0, sem_e1, sem_e2)
        sem_g = (sem_g0, sem_g1, sem_g2)
        sem_s = (sem_s0, sem_s1, sem_s2)

        pltpu.sync_copy(t_hbm, t_v)
        pltpu.sync_copy(q_hbm, q_v)

        @pl.loop(0, n, step=L)
        def _(i):
            den_v[0, pl.ds(i, L)] = jnp.zeros((L,), jnp.float32)

        @pl.loop(0, CH)
        def _(r):
            for cc in range(d // L):
                rows_v[0, r, pl.ds(cc * L, L)] = jnp.zeros((L,), jnp.float32)

        base = s * rows_per_tile
        for k in range(nfull):
            pltpu.sync_copy(rows_v.at[0], num_s.at[pl.ds(base + k * CH, CH)])
        if rem:
            pltpu.sync_copy(rows_v.at[0, pl.ds(0, rem)],
                            num_s.at[pl.ds(base + nfull * CH, rem)])
        plsc.subcore_barrier()

        for b in range(NB):
            pltpu.async_copy(edges_hbm.at[wid, b], eidx_v.at[b], sem_e[b])
        pltpu.make_async_copy(edges_hbm.at[wid, 0], eidx_v.at[0], sem_e[0]).wait()
        pltpu.async_copy(z_hbm.at[eidx_v.at[0, 0]], rows_v.at[0], sem_g[0])

        def run_pipeline(nch, ebase):
            @pl.loop(0, nch // NB)
            def _(jj):
                _chunk_block(jj, nch, ebase)

        def _chunk_block(jj, nch, ebase):
            for b in range(NB):
                j = jj * NB + b
                b1 = (b + 1) % NB
                b2 = (b + 2) % NB

                pltpu.make_async_copy(
                    z_hbm.at[eidx_v.at[b, 0]], rows_v.at[b], sem_g[b]).wait()

                @pl.when(j + 1 < nch)
                def _():
                    @pl.when(j >= 2)
                    def _():
                        pltpu.make_async_copy(
                            rows_v.at[b1], num_s.at[dsts_v.at[b1]],
                            sem_s[b1]).wait()
                    pltpu.make_async_copy(
                        edges_hbm.at[wid, j + 1], eidx_v.at[b1],
                        sem_e[b1]).wait()
                    pltpu.async_copy(
                        z_hbm.at[eidx_v.at[b1, 0]], rows_v.at[b1], sem_g[b1])

                for cc in range(CH // L):
                    sl = pl.ds(cc * L, L)
                    sv = eidx_v[b, 0, sl]
                    dv = eidx_v[b, 1, sl]
                    qs = plsc.load_gather(q_v, [sv])
                    ts = plsc.load_gather(t_v, [sv])
                    td = plsc.load_gather(t_v, [dv])
                    u = jnp.exp(qs * (-jnp.abs(ts - td)) / 500.0)
                    w = jnp.exp(u)
                    gidx = ebase + j * CH + cc * L + lax.iota(jnp.int32, L)
                    w = jnp.where(gidx < e, w, 0.0)
                    w_v[sl] = w
                    dsts_v[b, sl] = dv
                    plsc.addupdate_scatter(
                        den_v, [jnp.zeros((L,), jnp.int32), dv], w)

                @pl.loop(0, CH, unroll=2)
                def _(r):
                    wsp = plsc.load_gather(w_v, [jnp.full((L,), r, jnp.int32)])
                    for cc in range(d // L):
                        sl = pl.ds(cc * L, L)
                        rows_v[b, r, sl] = rows_v[b, r, sl] * wsp

                pltpu.async_copy(rows_v.at[b], num_s.at[dsts_v.at[b]],
                                 sem_s[b], add=True)

                @pl.when((j >= 1) & (j + 2 < nch))
                def _():
                    pltpu.async_copy(edges_hbm.at[wid, j + 2], eidx_v.at[b2],
                                     sem_e[b2])

        e0w = nch0 * CH
        e1w = nch1 * CH

        @pl.when(c == 0)
        def _():
            run_pipeline(nch0, s * e0w)

        @pl.when(c == 1)
        def _():
            run_pipeline(nch1, NS * e0w + s * e1w)

        for b in range(NB):
            pltpu.make_async_copy(
                rows_v.at[b], num_s.at[dsts_v.at[b]], sem_s[b]).wait()

        plsc.subcore_barrier()

        pltpu.sync_copy(den_v, den_hbm.at[wid])
        for k in range(nfull):
            pltpu.sync_copy(num_s.at[pl.ds(base + k * CH, CH)],
                            num_hbm.at[c, pl.ds(base + k * CH, CH)])
        if rem:
            pltpu.sync_copy(num_s.at[pl.ds(base + nfull * CH, rem)],
                            num_hbm.at[c, pl.ds(base + nfull * CH, rem)])

    return pl.kernel(
        body,
        out_type=(
            jax.ShapeDtypeStruct((NC, np_, d), jnp.float32),
            jax.ShapeDtypeStruct((NW, 1, n), jnp.float32),
        ),
        mesh=mesh,
        compiler_params=pltpu.CompilerParams(needs_layout_passes=False),
        scratch_types=[
            pltpu.VMEM((n,), jnp.float32),
            pltpu.VMEM((n,), jnp.float32),
            pltpu.VMEM((NB, 2, CH), jnp.int32),
            pltpu.VMEM((CH,), jnp.float32),
            pltpu.VMEM((NB, CH), jnp.int32),
            pltpu.VMEM((NB, CH, d), jnp.float32),
            pltpu.VMEM((1, n), jnp.float32),
            pltpu.VMEM_SHARED((np_, d), jnp.float32),
        ] + [pltpu.SemaphoreType.DMA] * (3 * NB),
    )


def kernel(x, t, W_fc, w_t, edge_index):
    with _jax_config.enable_x64(False):
        n, d = x.shape
        e = edge_index.shape[1]

        f0 = 0.49
        nch0 = max(NB, int(round(e * f0 / (NS * CH * NB))) * NB)
        rem_e = max(NS * CH * NB, e - NS * nch0 * CH)
        nch1 = -(-rem_e // (NS * CH))
        nch1 = -(-nch1 // NB) * NB
        nchm = max(nch0, nch1)
        e0 = NS * nch0 * CH
        ep = e0 + NS * nch1 * CH

        ei = edge_index.astype(jnp.int32)
        sflat = jnp.pad(ei[0], (0, ep - e))
        dflat = jnp.pad(ei[1], (0, ep - e))

        def _shape_core(a):
            a0 = jnp.pad(a[:e0].reshape(NS, nch0, CH),
                         ((0, 0), (0, nchm - nch0), (0, 0)))
            a1 = jnp.pad(a[e0:].reshape(NS, nch1, CH),
                         ((0, 0), (0, nchm - nch1), (0, 0)))
            return jnp.concatenate([a0, a1], axis=0)

        edges = jnp.stack([_shape_core(sflat), _shape_core(dflat)],
                          axis=2)

        bn = 2000 if n % 2000 == 0 else n
        z, q2 = _zq_call(x, W_fc, w_t.reshape(d, 1), bn)
        q = q2.reshape(n)

        num, den = _make_sc_kernel(n, d, e, nch0, nch1)(t, q, edges, z)
        return _combine_call(z, num[:, :n, :], den.reshape(NW, n).T, bn)


# speedup vs baseline: 1.3705x; 1.1370x over previous
import jax
import jax.numpy as jnp
from jax import lax
from jax._src import config as _jax_config
from jax.experimental import pallas as pl
from jax.experimental.pallas import tpu as pltpu
from jax.experimental.pallas import tpu_sc as plsc

NC = 2
NS = 16
NW = NC * NS
L = 16
CH = 48
NB = 3


def _zq_body(x_ref, w_ref, wt_ref, z_ref, q_ref):
    z = lax.dot_general(
        x_ref[...], w_ref[...], (((1,), (1,)), ((), ())),
        preferred_element_type=jnp.float32)
    z_ref[...] = z
    q_ref[...] = lax.dot_general(
        z, wt_ref[...], (((1,), (0,)), ((), ())),
        preferred_element_type=jnp.float32)


def _zq_call(x, W_fc, wt2, bn):
    n, d = x.shape
    return pl.pallas_call(
        _zq_body,
        grid=(n // bn,),
        in_specs=[
            pl.BlockSpec((bn, d), lambda i: (i, 0)),
            pl.BlockSpec((d, d), lambda i: (0, 0)),
            pl.BlockSpec((d, 1), lambda i: (0, 0)),
        ],
        out_specs=[
            pl.BlockSpec((bn, d), lambda i: (i, 0)),
            pl.BlockSpec((bn, 1), lambda i: (i, 0)),
        ],
        out_shape=[
            jax.ShapeDtypeStruct((n, d), jnp.float32),
            jax.ShapeDtypeStruct((n, 1), jnp.float32),
        ],
    )(x, W_fc, wt2)


def _combine_body(z_ref, num_ref, den_ref, o_ref):
    den = jnp.sum(den_ref[...], axis=1)
    num = num_ref[0] + num_ref[1]
    dcol = den[:, None]
    h = jnp.where(dcol > 0.0, num / jnp.where(dcol > 0.0, dcol, 1.0), 0.0)
    o_ref[...] = z_ref[...] + h


def _combine_call(z, num, den, bn):
    n, d = z.shape
    return pl.pallas_call(
        _combine_body,
        grid=(n // bn,),
        in_specs=[
            pl.BlockSpec((bn, d), lambda i: (i, 0)),
            pl.BlockSpec((NC, bn, d), lambda i: (0, i, 0)),
            pl.BlockSpec((bn, NW), lambda i: (i, 0)),
        ],
        out_specs=pl.BlockSpec((bn, d), lambda i: (i, 0)),
        out_shape=jax.ShapeDtypeStruct((n, d), jnp.float32),
    )(z, num, den)


def _make_sc_kernel(n, d, e, nch0, nch1):
    rows_per_tile = (-(-n // NS) + 7) // 8 * 8
    np_ = NS * rows_per_tile
    nfull = rows_per_tile // CH
    rem = rows_per_tile - nfull * CH
    mesh = plsc.VectorSubcoreMesh(
        core_axis_name="c", subcore_axis_name="s",
        num_cores=NC, num_subcores=NS)

    def body(t_hbm, q_hbm, edges_hbm, z_hbm,
             num_hbm, den_hbm,
             t_v, q_v, eidx_v, w_v, dsts_v, rows_v, den_v, num_s,
             sem_e0, sem_e1, sem_e2, sem_g0, sem_g1, sem_g2,
             sem_s0, sem_s1, sem_s2):
        c = lax.axis_index("c")
        s = lax.axis_index("s")
        wid = c * NS + s
        sem_e = (sem_e0, sem_e1, sem_e2)
        sem_g = (sem_g0, sem_g1, sem_g2)
        sem_s = (sem_s0, sem_s1, sem_s2)

        pltpu.sync_copy(t_hbm, t_v)
        pltpu.sync_copy(q_hbm, q_v)

        @pl.loop(0, n, step=L)
        def _(i):
            den_v[0, pl.ds(i, L)] = jnp.zeros((L,), jnp.float32)

        @pl.loop(0, CH)
        def _(r):
            for cc in range(d // L):
                rows_v[0, r, pl.ds(cc * L, L)] = jnp.zeros((L,), jnp.float32)

        base = s * rows_per_tile
        for k in range(nfull):
            pltpu.sync_copy(rows_v.at[0], num_s.at[pl.ds(base + k * CH, CH)])
        if rem:
            pltpu.sync_copy(rows_v.at[0, pl.ds(0, rem)],
                            num_s.at[pl.ds(base + nfull * CH, rem)])
        plsc.subcore_barrier()

        ep = NS * (nch0 + nch1) * CH

        def load_edges(slot, start):
            pltpu.async_copy(edges_hbm.at[pl.ds(start, CH)],
                             eidx_v.at[slot, 0], sem_e[slot])
            pltpu.async_copy(edges_hbm.at[pl.ds(ep + start, CH)],
                             eidx_v.at[slot, 1], sem_e[slot])

        def wait_edges(slot, start):
            pltpu.make_async_copy(edges_hbm.at[pl.ds(start, CH)],
                                  eidx_v.at[slot, 0], sem_e[slot]).wait()
            pltpu.make_async_copy(edges_hbm.at[pl.ds(ep + start, CH)],
                                  eidx_v.at[slot, 1], sem_e[slot]).wait()

        def run_pipeline(nch, ebase):
            for b in range(NB):
                load_edges(b, ebase + b * CH)
            wait_edges(0, ebase)
            pltpu.async_copy(z_hbm.at[eidx_v.at[0, 0]], rows_v.at[0], sem_g[0])

            @pl.loop(0, nch // NB)
            def _(jj):
                _chunk_block(jj, nch, ebase)

        def _chunk_block(jj, nch, ebase):
            for b in range(NB):
                j = jj * NB + b
                b1 = (b + 1) % NB
                b2 = (b + 2) % NB

                pltpu.make_async_copy(
                    z_hbm.at[eidx_v.at[b, 0]], rows_v.at[b], sem_g[b]).wait()

                @pl.when(j + 1 < nch)
                def _():
                    @pl.when(j >= 2)
                    def _():
                        pltpu.make_async_copy(
                            rows_v.at[b1], num_s.at[dsts_v.at[b1]],
                            sem_s[b1]).wait()
                    wait_edges(b1, ebase + (j + 1) * CH)
                    pltpu.async_copy(
                        z_hbm.at[eidx_v.at[b1, 0]], rows_v.at[b1], sem_g[b1])

                for cc in range(CH // L):
                    sl = pl.ds(cc * L, L)
                    sv = eidx_v[b, 0, sl]
                    dv = eidx_v[b, 1, sl]
                    qs = plsc.load_gather(q_v, [sv])
                    ts = plsc.load_gather(t_v, [sv])
                    td = plsc.load_gather(t_v, [dv])
                    u = jnp.exp(qs * (-jnp.abs(ts - td)) / 500.0)
                    w = jnp.exp(u)
                    gidx = ebase + j * CH + cc * L + lax.iota(jnp.int32, L)
                    w = jnp.where(gidx < e, w, 0.0)
                    w_v[sl] = w
                    dsts_v[b, sl] = dv
                    plsc.addupdate_scatter(
                        den_v, [jnp.zeros((L,), jnp.int32), dv], w)

                @pl.loop(0, CH, unroll=2)
                def _(r):
                    wsp = plsc.load_gather(w_v, [jnp.full((L,), r, jnp.int32)])
                    for cc in range(d // L):
                        sl = pl.ds(cc * L, L)
                        rows_v[b, r, sl] = rows_v[b, r, sl] * wsp

                pltpu.async_copy(rows_v.at[b], num_s.at[dsts_v.at[b]],
                                 sem_s[b], add=True)

                @pl.when((j >= 1) & (j + 2 < nch))
                def _():
                    load_edges(b2, ebase + (j + 2) * CH)

        e0w = nch0 * CH
        e1w = nch1 * CH

        @pl.when(c == 0)
        def _():
            run_pipeline(nch0, s * e0w)

        @pl.when(c == 1)
        def _():
            run_pipeline(nch1, NS * e0w + s * e1w)

        for b in range(NB):
            pltpu.make_async_copy(
                rows_v.at[b], num_s.at[dsts_v.at[b]], sem_s[b]).wait()

        plsc.subcore_barrier()

        pltpu.sync_copy(den_v, den_hbm.at[wid])
        for k in range(nfull):
            pltpu.sync_copy(num_s.at[pl.ds(base + k * CH, CH)],
                            num_hbm.at[c, pl.ds(base + k * CH, CH)])
        if rem:
            pltpu.sync_copy(num_s.at[pl.ds(base + nfull * CH, rem)],
                            num_hbm.at[c, pl.ds(base + nfull * CH, rem)])

    return pl.kernel(
        body,
        out_type=(
            jax.ShapeDtypeStruct((NC, np_, d), jnp.float32),
            jax.ShapeDtypeStruct((NW, 1, n), jnp.float32),
        ),
        mesh=mesh,
        compiler_params=pltpu.CompilerParams(needs_layout_passes=False),
        scratch_types=[
            pltpu.VMEM((n,), jnp.float32),
            pltpu.VMEM((n,), jnp.float32),
            pltpu.VMEM((NB, 2, CH), jnp.int32),
            pltpu.VMEM((CH,), jnp.float32),
            pltpu.VMEM((NB, CH), jnp.int32),
            pltpu.VMEM((NB, CH, d), jnp.float32),
            pltpu.VMEM((1, n), jnp.float32),
            pltpu.VMEM_SHARED((np_, d), jnp.float32),
        ] + [pltpu.SemaphoreType.DMA] * (3 * NB),
    )


def kernel(x, t, W_fc, w_t, edge_index):
    with _jax_config.enable_x64(False):
        n, d = x.shape
        e = edge_index.shape[1]

        f0 = 0.49
        nch0 = max(NB, int(round(e * f0 / (NS * CH * NB))) * NB)
        rem_e = max(NS * CH * NB, e - NS * nch0 * CH)
        nch1 = -(-rem_e // (NS * CH))
        nch1 = -(-nch1 // NB) * NB
        ep = NS * (nch0 + nch1) * CH
        edges = jnp.pad(edge_index.astype(jnp.int32),
                        ((0, 0), (0, ep - e))).reshape(2 * ep)

        bn = 2000 if n % 2000 == 0 else n
        z, q2 = _zq_call(x, W_fc, w_t.reshape(d, 1), bn)
        q = q2.reshape(n)

        num, den = _make_sc_kernel(n, d, e, nch0, nch1)(t, q, edges, z)
        return _combine_call(z, num[:, :n, :], den.reshape(NW, n).T, bn)
